# baseline (speedup 1.0000x reference)
import functools

import jax
import jax.numpy as jnp
from jax import lax
from jax.experimental import pallas as pl
from jax.experimental.pallas import tpu as pltpu
from jax.experimental.pallas import tpu_sc as plsc

B = 4096
L = 50
F = 26
D = 32

_info = plsc.get_sparse_core_info()
NC, NS = _info.num_cores, _info.num_subcores
NW = NC * NS
ROWS_PER_W = B // NW
NB = 32
N_SB = ROWS_PER_W // NB

_mesh = plsc.VectorSubcoreMesh(core_axis_name="c", subcore_axis_name="s")


@functools.partial(
    pl.kernel,
    mesh=_mesh,
    out_type=jax.ShapeDtypeStruct((B, D + F * D), jnp.float32),
    scratch_types=[
        pltpu.VMEM((NB * L,), jnp.int32),
        pltpu.VMEM((NB * L, D), jnp.float32),
        pltpu.VMEM((NB, D), jnp.float32),
        pltpu.VMEM((ROWS_PER_W,), jnp.int32),
        pltpu.VMEM((ROWS_PER_W, D), jnp.float32),
        pltpu.SemaphoreType.DMA,
    ],
    compiler_params=pltpu.CompilerParams(use_tc_tiling_on_sc=False),
)
def _sc_kernel(seq_flat, ctx_flat, item_table, ctx_table, out,
               item_idx_v, item_rows_v, seq_buf, ctx_idx_v, ctx_rows_v, sem):
    wid = lax.axis_index("s") * NC + lax.axis_index("c")
    base = wid * ROWS_PER_W

    for f in range(F):
        pltpu.sync_copy(ctx_flat.at[pl.ds(f * B + base, ROWS_PER_W)], ctx_idx_v)
        pltpu.async_copy(ctx_table.at[ctx_idx_v], ctx_rows_v, sem).wait()
        pltpu.sync_copy(
            ctx_rows_v,
            out.at[pl.ds(base, ROWS_PER_W), pl.ds(D + f * D, D)],
        )

    for sb in range(N_SB):
        sbase = base + sb * NB
        pltpu.sync_copy(seq_flat.at[pl.ds(sbase * L, NB * L)], item_idx_v)
        pltpu.async_copy(item_table.at[item_idx_v], item_rows_v, sem).wait()

        def body(b, carry):
            acc0 = jnp.zeros((16,), jnp.float32)
            acc1 = jnp.zeros((16,), jnp.float32)
            r0 = b * L
            for l in range(L):
                acc0 = acc0 + item_rows_v[r0 + l, pl.ds(0, 16)]
                acc1 = acc1 + item_rows_v[r0 + l, pl.ds(16, 16)]
            seq_buf[b, pl.ds(0, 16)] = acc0 * (1.0 / L)
            seq_buf[b, pl.ds(16, 16)] = acc1 * (1.0 / L)
            return carry

        lax.fori_loop(0, NB, body, 0)
        pltpu.sync_copy(seq_buf, out.at[pl.ds(sbase, NB), pl.ds(0, D)])


def kernel(item_seq, context_ids, item_table, context_table):
    seq_flat = item_seq.reshape(-1)
    ctx_flat = context_ids.T.reshape(-1)
    return _sc_kernel(seq_flat, ctx_flat, item_table, context_table)


# speedup vs baseline: 1.0536x; 1.0536x over previous
import functools

import jax
import jax.numpy as jnp
from jax import lax
from jax.experimental import pallas as pl
from jax.experimental.pallas import tpu as pltpu
from jax.experimental.pallas import tpu_sc as plsc

B = 4096
L = 50
F = 26
D = 32

_info = plsc.get_sparse_core_info()
NC, NS = _info.num_cores, _info.num_subcores
NW = NC * NS
RW = B // NW
NB = 16
N_SB = RW // NB
CBUF = 4

_mesh = plsc.VectorSubcoreMesh(core_axis_name="c", subcore_axis_name="s")


@functools.partial(
    pl.kernel,
    mesh=_mesh,
    out_type=jax.ShapeDtypeStruct((B, D + F * D), jnp.float32),
    scratch_types=[
        pltpu.VMEM((RW * L,), jnp.int32),
        pltpu.VMEM((F, RW), jnp.int32),
        pltpu.VMEM((NB * L, D), jnp.float32),
        pltpu.VMEM((NB * L, D), jnp.float32),
        pltpu.VMEM((NB, D), jnp.float32),
        pltpu.VMEM((NB, D), jnp.float32),
        pltpu.VMEM((RW, D), jnp.float32),
        pltpu.VMEM((RW, D), jnp.float32),
        pltpu.VMEM((RW, D), jnp.float32),
        pltpu.VMEM((RW, D), jnp.float32),
        pltpu.SemaphoreType.DMA,
        pltpu.SemaphoreType.DMA,
        pltpu.SemaphoreType.DMA,
        pltpu.SemaphoreType.DMA,
        pltpu.SemaphoreType.DMA,
        pltpu.SemaphoreType.DMA,
    ],
    compiler_params=pltpu.CompilerParams(use_tc_tiling_on_sc=False),
)
def _sc_kernel(seq_flat, ctx_t, item_table, ctx_table, out,
               item_idx_v, ctx_idx_v, ir0, ir1, sb0, sb1,
               cb0, cb1, cb2, cb3,
               isem, icsem, gsem, cgsem, ssem, csem):
    wid = lax.axis_index("s") * NC + lax.axis_index("c")
    base = wid * RW

    cp_i = pltpu.async_copy(seq_flat.at[pl.ds(base * L, RW * L)], item_idx_v, isem)
    cp_c = pltpu.async_copy(ctx_t.at[:, pl.ds(base, RW)], ctx_idx_v, icsem)
    cp_c.wait()

    cbufs = [cb0, cb1, cb2, cb3]
    cg = [None] * F
    cs = [None] * F

    def ctx_store(f):
        return pltpu.async_copy(
            cbufs[f % CBUF],
            out.at[pl.ds(base, RW), pl.ds(D + f * D, D)],
            csem,
        )

    for f in range(F):
        if f >= CBUF:
            cs[f - CBUF].wait()
        cg[f] = pltpu.async_copy(
            ctx_table.at[ctx_idx_v.at[f]], cbufs[f % CBUF], cgsem)
        if f >= 1:
            cg[f - 1].wait()
            cs[f - 1] = ctx_store(f - 1)
    cg[F - 1].wait()
    cs[F - 1] = ctx_store(F - 1)

    cp_i.wait()
    irows = [ir0, ir1]
    sbufs = [sb0, sb1]
    ig = [None] * N_SB
    st = [None] * N_SB

    def reduce_block(rows_ref, sbuf_ref):
        def body(b, carry):
            acc0 = jnp.zeros((16,), jnp.float32)
            acc1 = jnp.zeros((16,), jnp.float32)
            r0 = b * L
            for l in range(L):
                acc0 = acc0 + rows_ref[r0 + l, pl.ds(0, 16)]
                acc1 = acc1 + rows_ref[r0 + l, pl.ds(16, 16)]
            sbuf_ref[b, pl.ds(0, 16)] = acc0 * (1.0 / L)
            sbuf_ref[b, pl.ds(16, 16)] = acc1 * (1.0 / L)
            return carry

        lax.fori_loop(0, NB, body, 0)

    def item_step(p):
        if p >= 2:
            st[p - 2].wait()
        ig[p].wait()
        reduce_block(irows[p % 2], sbufs[p % 2])
        st[p] = pltpu.async_copy(
            sbufs[p % 2],
            out.at[pl.ds(base + p * NB, NB), pl.ds(0, D)],
            ssem,
        )

    for sb in range(N_SB):
        ig[sb] = pltpu.async_copy(
            item_table.at[item_idx_v.at[pl.ds(sb * NB * L, NB * L)]],
            irows[sb % 2], gsem)
        if sb >= 1:
            item_step(sb - 1)
    item_step(N_SB - 1)

    for f in range(F - CBUF, F):
        cs[f].wait()
    st[N_SB - 2].wait()
    st[N_SB - 1].wait()


def kernel(item_seq, context_ids, item_table, context_table):
    seq_flat = item_seq.reshape(-1)
    ctx_t = context_ids.T
    return _sc_kernel(seq_flat, ctx_t, item_table, context_table)
